# baseline (speedup 1.0000x reference)
import functools

import jax
import jax.numpy as jnp
from jax import lax
from jax.experimental import pallas as pl
from jax.experimental.pallas import tpu as pltpu
from jax.experimental.pallas import tpu_sc as plsc

B = 64
N = 32768
D = 8
KTOP = 3
NC, NS, L = 2, 16, 16
NW = NC * NS
SC_B = 32
TC_B = B - SC_B
ROW = N * D
RUN = 128
GROUP = RUN * D
CHUNK = 16384
GROUPS_PER_CHUNK = CHUNK // GROUP
NCHUNK = ROW // CHUNK
PAIRS = NCHUNK // 2
NBLK = N // RUN


def _insert3(m1, m2, m3, v):
    t1 = jnp.minimum(m1, v)
    m1 = jnp.maximum(m1, v)
    t2 = jnp.minimum(m2, t1)
    m2 = jnp.maximum(m2, t1)
    m3 = jnp.maximum(m3, t2)
    return m1, m2, m3


def _merge_triples(m1, m2, m3, h1, h2, h3):
    t1 = jnp.minimum(m1, h1)
    m1 = jnp.maximum(m1, h1)
    t2 = jnp.minimum(m2, t1)
    m2 = jnp.maximum(m2, t1)
    m3 = jnp.maximum(m3, t2)
    t = jnp.minimum(m2, h2)
    m2 = jnp.maximum(m2, h2)
    m3 = jnp.maximum(m3, jnp.maximum(t, h3))
    return m1, m2, m3


def _bf16_rne(x):
    u = lax.bitcast_convert_type(x, jnp.int32)
    lsb = jnp.bitwise_and(jnp.right_shift(u, 16), 1)
    r = jnp.bitwise_and(u + 32767 + lsb, jnp.int32(-65536))
    return lax.bitcast_convert_type(r, jnp.float32)


def _worker_id():
    return lax.axis_index("s") * NC + lax.axis_index("c")


def _sc_body(x_hbm, coef_hbm, out_hbm, buf_a, buf_b, coef_v, out_v,
             sem_a, sem_b, sem_c):
    wid = _worker_id()
    base = wid * ROW

    pltpu.async_copy(coef_hbm, coef_v, sem_c).wait()

    def dma_start(ci, buf, sem):
        pltpu.make_async_copy(
            x_hbm.at[pl.ds(base + ci * CHUNK, CHUNK)], buf, sem).start()

    def dma_wait(buf, sem):
        pltpu.make_async_copy(
            x_hbm.at[pl.ds(0, CHUNK)], buf, sem).wait()

    dma_start(0, buf_a, sem_a)
    dma_start(1, buf_b, sem_b)

    neg_inf = jnp.full((L,), -jnp.inf, dtype=jnp.float32)
    acc0 = tuple(neg_inf for _ in range(3 * D))

    def process(buf, acc):
        def gbody(i, acc):
            accl = list(acc)
            off = i * GROUP
            for c in range(D):
                m1, m2, m3 = accl[3 * c], accl[3 * c + 1], accl[3 * c + 2]
                for u in range(RUN // L):
                    v = buf[pl.ds(off + c * RUN + u * L, L)]
                    m1, m2, m3 = _insert3(m1, m2, m3, v)
                accl[3 * c], accl[3 * c + 1], accl[3 * c + 2] = m1, m2, m3
            return tuple(accl)
        return lax.fori_loop(0, GROUPS_PER_CHUNK, gbody, acc)

    def pair_body(p, acc):
        ci = p * 2
        dma_wait(buf_a, sem_a)
        acc = process(buf_a, acc)

        @pl.when(ci + 2 < NCHUNK)
        def _():
            dma_start(ci + 2, buf_a, sem_a)

        dma_wait(buf_b, sem_b)
        acc = process(buf_b, acc)

        @pl.when(ci + 3 < NCHUNK)
        def _():
            dma_start(ci + 3, buf_b, sem_b)

        return acc

    acc = lax.fori_loop(0, PAIRS, pair_body, acc0)

    lanes = lax.iota(jnp.int32, L)

    def top3_all_lanes(m1, m2, m3):
        for sh in (8, 4, 2, 1):
            idx = jnp.bitwise_xor(lanes, sh)
            h1 = jnp.take_along_axis(m1, idx, axis=0)
            h2 = jnp.take_along_axis(m2, idx, axis=0)
            h3 = jnp.take_along_axis(m3, idx, axis=0)
            m1, m2, m3 = _merge_triples(m1, m2, m3, h1, h2, h3)
        return m1, m2, m3

    zero = jnp.zeros((L,), jnp.float32)
    M1 = M2 = M3 = zero
    for c in range(D):
        m1, m2, m3 = top3_all_lanes(acc[3 * c], acc[3 * c + 1],
                                    acc[3 * c + 2])
        sel = lanes == c
        M1 = jnp.where(sel, m1, M1)
        M2 = jnp.where(sel, m2, M2)
        M3 = jnp.where(sel, m3, M3)

    Fm = ((M1 + M2) + M3) / 3.0
    s = _bf16_rne(M1) * _bf16_rne(coef_v[0]) + \
        _bf16_rne(M2) * _bf16_rne(coef_v[1]) + \
        _bf16_rne(M3) * _bf16_rne(coef_v[2]) + \
        _bf16_rne(Fm) * _bf16_rne(coef_v[3])
    for sh in (4, 2, 1):
        s = s + jnp.take_along_axis(s, jnp.bitwise_xor(lanes, sh), axis=0)

    ov = jnp.where(lanes == 0, s, zero) + coef_v[4]
    out_v[:] = ov
    pltpu.async_copy(out_v, out_hbm.at[wid], sem_c).wait()


def _launch_sc(x_flat, coef):
    mesh = plsc.VectorSubcoreMesh(core_axis_name="c", subcore_axis_name="s",
                                  num_cores=NC, num_subcores=NS)
    return pl.kernel(
        _sc_body,
        out_type=jax.ShapeDtypeStruct((NW, L), jnp.float32),
        mesh=mesh,
        scratch_types=[
            pltpu.VMEM((CHUNK,), jnp.float32),
            pltpu.VMEM((CHUNK,), jnp.float32),
            pltpu.VMEM((5, L), jnp.float32),
            pltpu.VMEM((L,), jnp.float32),
            pltpu.SemaphoreType.DMA,
            pltpu.SemaphoreType.DMA,
            pltpu.SemaphoreType.DMA,
        ],
    )(x_flat, coef)


def _tc_body(xv_ref, coef_ref, out_ref):
    TUF = 16
    neg_inf = jnp.full((TUF, D, RUN), -jnp.inf, dtype=jnp.float32)

    def tbody(t, carry):
        return _insert3(*carry, xv_ref[0, pl.ds(t * TUF, TUF)])

    m1, m2, m3 = lax.fori_loop(0, NBLK // TUF, tbody,
                               (neg_inf, neg_inf, neg_inf))

    trips = [(m1[j], m2[j], m3[j]) for j in range(TUF)]
    while len(trips) > 1:
        trips = [_merge_triples(*a, *b)
                 for a, b in zip(trips[::2], trips[1::2])]
    m1, m2, m3 = trips[0]

    for sh in (64, 32, 16, 8, 4, 2, 1):
        h1 = pltpu.roll(m1, sh, 1)
        h2 = pltpu.roll(m2, sh, 1)
        h3 = pltpu.roll(m3, sh, 1)
        m1, m2, m3 = _merge_triples(m1, m2, m3, h1, h2, h3)

    t1 = m1[:, 0:1]
    t2 = m2[:, 0:1]
    t3 = m3[:, 0:1]
    mean = ((t1 + t2) + t3) / 3.0

    s8 = _bf16_rne(t1) * _bf16_rne(coef_ref[:, 0:1]) + \
         _bf16_rne(t2) * _bf16_rne(coef_ref[:, 1:2]) + \
         _bf16_rne(t3) * _bf16_rne(coef_ref[:, 2:3]) + \
         _bf16_rne(mean) * _bf16_rne(coef_ref[:, 3:4])
    out_ref[0, 0, :] = jnp.full((RUN,), jnp.sum(s8) + coef_ref[0, 4],
                                jnp.float32)


def _launch_tc(xv4, coef_tc):
    return pl.pallas_call(
        _tc_body,
        grid=(TC_B,),
        in_specs=[
            pl.BlockSpec((1, NBLK, D, RUN), lambda i: (i + SC_B, 0, 0, 0)),
            pl.BlockSpec((D, 5), lambda i: (0, 0)),
        ],
        out_specs=pl.BlockSpec((1, 1, RUN), lambda i: (i, 0, 0)),
        out_shape=jax.ShapeDtypeStruct((TC_B, 1, RUN), jnp.float32),
    )(xv4, coef_tc)


def _view4(x):
    return x.reshape(B, NBLK, RUN, D).transpose(0, 1, 3, 2)


def _make_coefs(W, b):
    Wt = W.reshape(D, KTOP + 1)
    w_pad = jnp.concatenate([Wt, jnp.zeros_like(Wt)], axis=0)
    coef_sc = jnp.concatenate(
        [w_pad.T, jnp.full((1, L), b[0], jnp.float32)], axis=0)
    coef_tc = jnp.concatenate(
        [Wt, jnp.full((D, 1), b[0], jnp.float32)], axis=1)
    return coef_sc, coef_tc


@jax.jit
def _run(x, W, b):
    xv4 = _view4(x)
    coef_sc, coef_tc = _make_coefs(W, b)
    out_sc = _launch_sc(xv4.reshape(-1), coef_sc)
    out_tc = _launch_tc(xv4, coef_tc)
    return jnp.concatenate([out_sc[:, 0], out_tc[:, 0, 0]]).reshape(B, 1)


def kernel(x, W, b):
    return _run(x, W, b)


# speedup vs baseline: 1.0084x; 1.0084x over previous
import jax
import jax.numpy as jnp
from jax import lax
from jax.experimental import pallas as pl
from jax.experimental.pallas import tpu as pltpu
from jax.experimental.pallas import tpu_sc as plsc

B = 64
N = 32768
D = 8
KTOP = 3
NC, NS, L = 2, 16, 16
NW = NC * NS
SC_B = 32
TC_B = B - SC_B
ROW = N * D
RUN = 128
GROUP = RUN * D
CHUNK = 16384
GROUPS_PER_CHUNK = CHUNK // GROUP
NCHUNK = ROW // CHUNK
PAIRS = NCHUNK // 2
NBLK = N // RUN


def _insert3(m1, m2, m3, v):
    t1 = jnp.minimum(m1, v)
    m1 = jnp.maximum(m1, v)
    t2 = jnp.minimum(m2, t1)
    m2 = jnp.maximum(m2, t1)
    m3 = jnp.maximum(m3, t2)
    return m1, m2, m3


def _merge_triples(m1, m2, m3, h1, h2, h3):
    t1 = jnp.minimum(m1, h1)
    m1 = jnp.maximum(m1, h1)
    t2 = jnp.minimum(m2, t1)
    m2 = jnp.maximum(m2, t1)
    m3 = jnp.maximum(m3, t2)
    t = jnp.minimum(m2, h2)
    m2 = jnp.maximum(m2, h2)
    m3 = jnp.maximum(m3, jnp.maximum(t, h3))
    return m1, m2, m3


def _bf16_rne(x):
    u = lax.bitcast_convert_type(x, jnp.int32)
    lsb = jnp.bitwise_and(jnp.right_shift(u, 16), 1)
    r = jnp.bitwise_and(u + 32767 + lsb, jnp.int32(-65536))
    return lax.bitcast_convert_type(r, jnp.float32)


def _worker_id():
    return lax.axis_index("s") * NC + lax.axis_index("c")


def _sc_body(x_hbm, coef_hbm, out_hbm, buf_a, buf_b, coef_v, out_v,
             sem_a, sem_b, sem_c):
    wid = _worker_id()
    base = wid * ROW

    pltpu.async_copy(coef_hbm, coef_v, sem_c).wait()

    def dma_start(ci, buf, sem):
        pltpu.make_async_copy(
            x_hbm.at[pl.ds(base + ci * CHUNK, CHUNK)], buf, sem).start()

    def dma_wait(buf, sem):
        pltpu.make_async_copy(
            x_hbm.at[pl.ds(0, CHUNK)], buf, sem).wait()

    dma_start(0, buf_a, sem_a)
    dma_start(1, buf_b, sem_b)

    neg_inf = jnp.full((L,), -jnp.inf, dtype=jnp.float32)
    acc0 = tuple(neg_inf for _ in range(3 * D))

    def process(buf, acc):
        def gbody(i, acc):
            accl = list(acc)
            off = i * GROUP
            for c in range(D):
                m1, m2, m3 = accl[3 * c], accl[3 * c + 1], accl[3 * c + 2]
                for u in range(RUN // L):
                    v = buf[pl.ds(off + c * RUN + u * L, L)]
                    m1, m2, m3 = _insert3(m1, m2, m3, v)
                accl[3 * c], accl[3 * c + 1], accl[3 * c + 2] = m1, m2, m3
            return tuple(accl)
        return lax.fori_loop(0, GROUPS_PER_CHUNK, gbody, acc)

    def pair_body(p, acc):
        ci = p * 2
        dma_wait(buf_a, sem_a)
        acc = process(buf_a, acc)

        @pl.when(ci + 2 < NCHUNK)
        def _():
            dma_start(ci + 2, buf_a, sem_a)

        dma_wait(buf_b, sem_b)
        acc = process(buf_b, acc)

        @pl.when(ci + 3 < NCHUNK)
        def _():
            dma_start(ci + 3, buf_b, sem_b)

        return acc

    acc = lax.fori_loop(0, PAIRS, pair_body, acc0)

    lanes = lax.iota(jnp.int32, L)

    def top3_all_lanes(m1, m2, m3):
        for sh in (8, 4, 2, 1):
            idx = jnp.bitwise_xor(lanes, sh)
            h1 = jnp.take_along_axis(m1, idx, axis=0)
            h2 = jnp.take_along_axis(m2, idx, axis=0)
            h3 = jnp.take_along_axis(m3, idx, axis=0)
            m1, m2, m3 = _merge_triples(m1, m2, m3, h1, h2, h3)
        return m1, m2, m3

    zero = jnp.zeros((L,), jnp.float32)
    M1 = M2 = M3 = zero
    for c in range(D):
        m1, m2, m3 = top3_all_lanes(acc[3 * c], acc[3 * c + 1],
                                    acc[3 * c + 2])
        sel = lanes == c
        M1 = jnp.where(sel, m1, M1)
        M2 = jnp.where(sel, m2, M2)
        M3 = jnp.where(sel, m3, M3)

    Fm = ((M1 + M2) + M3) / 3.0
    s = _bf16_rne(M1) * _bf16_rne(coef_v[0]) + \
        _bf16_rne(M2) * _bf16_rne(coef_v[1]) + \
        _bf16_rne(M3) * _bf16_rne(coef_v[2]) + \
        _bf16_rne(Fm) * _bf16_rne(coef_v[3])
    for sh in (4, 2, 1):
        s = s + jnp.take_along_axis(s, jnp.bitwise_xor(lanes, sh), axis=0)

    ov = jnp.where(lanes == 0, s, zero) + coef_v[4]
    out_v[:] = ov
    pltpu.async_copy(out_v, out_hbm.at[wid], sem_c).wait()


def _launch_sc(x_flat, coef):
    mesh = plsc.VectorSubcoreMesh(core_axis_name="c", subcore_axis_name="s",
                                  num_cores=NC, num_subcores=NS)
    return pl.kernel(
        _sc_body,
        out_type=jax.ShapeDtypeStruct((NW, L), jnp.float32),
        mesh=mesh,
        scratch_types=[
            pltpu.VMEM((CHUNK,), jnp.float32),
            pltpu.VMEM((CHUNK,), jnp.float32),
            pltpu.VMEM((5, L), jnp.float32),
            pltpu.VMEM((L,), jnp.float32),
            pltpu.SemaphoreType.DMA,
            pltpu.SemaphoreType.DMA,
            pltpu.SemaphoreType.DMA,
        ],
    )(x_flat, coef)


def _tc_body(xv_ref, coef_ref, out_ref):
    TUF = 8
    neg_inf = jnp.full((TUF, D, RUN), -jnp.inf, dtype=jnp.float32)

    def tbody(t, carry):
        return _insert3(*carry, xv_ref[0, pl.ds(t * TUF, TUF)])

    m1, m2, m3 = lax.fori_loop(0, NBLK // TUF, tbody,
                               (neg_inf, neg_inf, neg_inf))

    trips = [(m1[j], m2[j], m3[j]) for j in range(TUF)]
    while len(trips) > 1:
        trips = [_merge_triples(*a, *b)
                 for a, b in zip(trips[::2], trips[1::2])]
    m1, m2, m3 = trips[0]

    for sh in (64, 32, 16, 8, 4, 2, 1):
        h1 = pltpu.roll(m1, sh, 1)
        h2 = pltpu.roll(m2, sh, 1)
        h3 = pltpu.roll(m3, sh, 1)
        m1, m2, m3 = _merge_triples(m1, m2, m3, h1, h2, h3)

    t1 = m1[:, 0:1]
    t2 = m2[:, 0:1]
    t3 = m3[:, 0:1]
    mean = ((t1 + t2) + t3) / 3.0

    s8 = _bf16_rne(t1) * _bf16_rne(coef_ref[:, 0:1]) + \
         _bf16_rne(t2) * _bf16_rne(coef_ref[:, 1:2]) + \
         _bf16_rne(t3) * _bf16_rne(coef_ref[:, 2:3]) + \
         _bf16_rne(mean) * _bf16_rne(coef_ref[:, 3:4])
    out_ref[0, 0, :] = jnp.full((RUN,), jnp.sum(s8) + coef_ref[0, 4],
                                jnp.float32)


def _launch_tc(xv4, coef_tc):
    return pl.pallas_call(
        _tc_body,
        grid=(TC_B,),
        in_specs=[
            pl.BlockSpec((1, NBLK, D, RUN), lambda i: (i + SC_B, 0, 0, 0)),
            pl.BlockSpec((D, 5), lambda i: (0, 0)),
        ],
        out_specs=pl.BlockSpec((1, 1, RUN), lambda i: (i, 0, 0)),
        out_shape=jax.ShapeDtypeStruct((TC_B, 1, RUN), jnp.float32),
    )(xv4, coef_tc)


def _view4(x):
    return x.reshape(B, NBLK, RUN, D).transpose(0, 1, 3, 2)


def _make_coefs(W, b):
    Wt = W.reshape(D, KTOP + 1)
    w_pad = jnp.concatenate([Wt, jnp.zeros_like(Wt)], axis=0)
    coef_sc = jnp.concatenate(
        [w_pad.T, jnp.full((1, L), b[0], jnp.float32)], axis=0)
    coef_tc = jnp.concatenate(
        [Wt, jnp.full((D, 1), b[0], jnp.float32)], axis=1)
    return coef_sc, coef_tc


@jax.jit
def _run(x, W, b):
    xv4 = _view4(x)
    coef_sc, coef_tc = _make_coefs(W, b)
    out_sc = _launch_sc(xv4.reshape(-1), coef_sc)
    out_tc = _launch_tc(xv4, coef_tc)
    return jnp.concatenate([out_sc[:, 0], out_tc[:, 0, 0]]).reshape(B, 1)


def kernel(x, W, b):
    return _run(x, W, b)
